# Initial kernel scaffold; baseline (speedup 1.0000x reference)
import jax
import jax.numpy as jnp
from jax.experimental import pallas as pl


def kernel(x, W1, A1, W2, A2, W3, W3ms, W3cat, W5):
    raise NotImplementedError("write your pallas kernel here")


# speedup vs baseline: 13.3248x; 13.3248x over previous
import functools

import jax
import jax.numpy as jnp
import numpy as np
from jax import lax
from jax.experimental import pallas as pl
from jax.experimental.pallas import tpu as pltpu
from jax.experimental.pallas import tpu_sc as plsc

_K = 16
_BN_S = 1.0 / np.sqrt(1.0 + 1e-5)
_NEG = -3.0e38


def _leaky(y):
    return jnp.where(y > 0, y, 0.2 * y)


def _knn_body(tab_ref, tabT_ref, idx_ref):
    b = pl.program_id(0)
    i = pl.program_id(1)
    xt = tab_ref[0]
    xT = tabT_ref[0]
    n = xT.shape[1]
    tn = xt.shape[0]
    scores = 2.0 * jnp.dot(xt, xT, preferred_element_type=jnp.float32)
    scores = scores - jnp.sum(xT * xT, axis=0, keepdims=True)
    lane = lax.broadcasted_iota(jnp.int32, (tn, n), 1)
    row = lax.broadcasted_iota(jnp.int32, (tn, n), 0) + i * tn
    scores = jnp.where(lane == row, _NEG, scores)
    col = lax.broadcasted_iota(jnp.int32, (tn, _K), 1)
    acc = jnp.zeros((tn, _K), jnp.int32)
    base = b * n
    for j in range(_K):
        m = jnp.max(scores, axis=1, keepdims=True)
        cand = jnp.where(scores == m, lane, n)
        sel = jnp.min(cand, axis=1)
        acc = jnp.where(col == j, (sel + base)[:, None], acc)
        scores = jnp.where(lane == sel[:, None], _NEG, scores)
    idx_ref[0] = acc


def _knn(tab, tabT, tn=256):
    B, N, C = tab.shape
    return pl.pallas_call(
        _knn_body,
        grid=(B, N // tn),
        in_specs=[
            pl.BlockSpec((1, tn, C), lambda b, i: (b, i, 0)),
            pl.BlockSpec((1, C, N), lambda b, i: (b, 0, 0)),
        ],
        out_specs=pl.BlockSpec((1, tn, _K), lambda b, i: (b, i, 0)),
        out_shape=jax.ShapeDtypeStruct((B, N, _K), jnp.int32),
    )(tab, tabT)


def _sc_gather(table, idx, ch):
    R, C = table.shape
    Rk = idx.shape[0]
    info = plsc.get_sparse_core_info()
    nc = info.num_cores
    nw = nc * info.num_subcores
    per_w = Rk // nw
    chunks = per_w // ch
    mesh = plsc.VectorSubcoreMesh(core_axis_name="c", subcore_axis_name="s")

    @functools.partial(
        pl.kernel, mesh=mesh,
        out_type=jax.ShapeDtypeStruct((Rk, C), jnp.float32),
        compiler_params=pltpu.CompilerParams(use_tc_tiling_on_sc=False),
        scratch_types=[
            pltpu.VMEM((ch,), jnp.int32),
            pltpu.VMEM((ch, C), jnp.float32),
            pltpu.SemaphoreType.DMA,
        ],
    )
    def k(table_hbm, idx_hbm, out_hbm, idx_v, rows_v, sem):
        wid = lax.axis_index("s") * nc + lax.axis_index("c")
        base = wid * per_w
        for c in range(chunks):
            off = base + c * ch
            pltpu.sync_copy(idx_hbm.at[pl.ds(off, ch)], idx_v)
            pltpu.async_copy(table_hbm.at[idx_v], rows_v, sem).wait()
            pltpu.sync_copy(rows_v, out_hbm.at[pl.ds(off, ch)])

    return k(table, idx)


def _att_body(nb_ref, xc_ref, W_ref, A_ref, out_ref, *, C, Cp):
    nb = nb_ref[0]
    xc = xc_ref[0]
    W = W_ref[...]
    A = A_ref[...]
    Wa, Wb = W[:, :C], W[:, C:]
    Aa, Ab = A[:, :C], A[:, C:]
    if Cp != C:
        z = jnp.zeros((64, Cp - C), jnp.float32)
        Wa = jnp.concatenate([Wa, z], axis=1)
        Wb = jnp.concatenate([Wb, z], axis=1)
        Aa = jnp.concatenate([Aa, z], axis=1)
        Ab = jnp.concatenate([Ab, z], axis=1)
    Wf = jnp.concatenate([Wa, Wb], axis=1)
    Af = jnp.concatenate([Aa, Ab], axis=1)
    tm = xc.shape[0]
    nb3 = nb.reshape(tm, _K, Cp)
    xcb = jnp.broadcast_to(xc[:, None, :], (tm, _K, Cp))
    d = nb3 - xcb
    featW = jnp.concatenate([d, xcb], axis=2).reshape(tm * _K, 2 * Cp)
    featA = jnp.concatenate([xcb - nb3, nb3], axis=2).reshape(tm * _K, 2 * Cp)
    nt = (((1,), (1,)), ((), ()))
    y = lax.dot_general(featW, Wf, nt, preferred_element_type=jnp.float32)
    ey = lax.dot_general(featA, Af, nt, preferred_element_type=jnp.float32)
    cpre = _leaky(y * _BN_S).reshape(tm, _K, 64)
    e = _leaky(ey * _BN_S).reshape(tm, _K, 64)
    m = jnp.max(e, axis=1, keepdims=True)
    w = jnp.exp(e - m)
    att = w / jnp.sum(w, axis=1, keepdims=True)
    out_ref[0] = jnp.sum(att * cpre, axis=1)


def _att(nb, tab, W, A, C, tm=256):
    B, N, Cp = tab.shape
    body = functools.partial(_att_body, C=C, Cp=Cp)
    return pl.pallas_call(
        body,
        grid=(B, N // tm),
        in_specs=[
            pl.BlockSpec((1, tm * _K, Cp), lambda b, i: (b, i, 0)),
            pl.BlockSpec((1, tm, Cp), lambda b, i: (b, i, 0)),
            pl.BlockSpec(W.shape, lambda b, i: (0, 0)),
            pl.BlockSpec(A.shape, lambda b, i: (0, 0)),
        ],
        out_specs=pl.BlockSpec((1, tm, 64), lambda b, i: (b, i, 0)),
        out_shape=jax.ShapeDtypeStruct((B, N, 64), jnp.float32),
    )(nb, tab, W, A)


def _s3_body(nb_ref, c2_ref, c1_ref, W3_ref, W3ms_ref, W3cat_ref, W5_ref,
             out_ref):
    nb = nb_ref[0]
    c2 = c2_ref[0]
    c1 = c1_ref[0]
    W3 = W3_ref[...]
    W3ms = W3ms_ref[...]
    Wst = jnp.concatenate([W3, W3ms], axis=0)
    tm = c2.shape[0]
    nb3 = nb.reshape(tm, _K, 64)
    c2b = jnp.broadcast_to(c2[:, None, :], (tm, _K, 64))
    feat = jnp.concatenate([nb3 - c2b, c2b], axis=2).reshape(tm * _K, 128)
    nt = (((1,), (1,)), ((), ()))
    G = lax.dot_general(feat, Wst, nt, preferred_element_type=jnp.float32)
    c3c = jnp.max(_leaky(G * _BN_S).reshape(tm, _K, 256), axis=1)
    c3cat = _leaky(lax.dot_general(
        c3c, W3cat_ref[...], nt, preferred_element_type=jnp.float32) * _BN_S)
    xall = jnp.concatenate([c1, c2, c3c[:, :128], c3cat], axis=1)
    out_ref[0] = _leaky(lax.dot_general(
        xall, W5_ref[...], nt, preferred_element_type=jnp.float32) * _BN_S)


def _s3(nb, c2, c1, W3, W3ms, W3cat, W5, tm=256):
    B, N, _ = c2.shape
    return pl.pallas_call(
        _s3_body,
        grid=(B, N // tm),
        in_specs=[
            pl.BlockSpec((1, tm * _K, 64), lambda b, i: (b, i, 0)),
            pl.BlockSpec((1, tm, 64), lambda b, i: (b, i, 0)),
            pl.BlockSpec((1, tm, 64), lambda b, i: (b, i, 0)),
            pl.BlockSpec(W3.shape, lambda b, i: (0, 0)),
            pl.BlockSpec(W3ms.shape, lambda b, i: (0, 0)),
            pl.BlockSpec(W3cat.shape, lambda b, i: (0, 0)),
            pl.BlockSpec(W5.shape, lambda b, i: (0, 0)),
        ],
        out_specs=pl.BlockSpec((1, tm, 512), lambda b, i: (b, i, 0)),
        out_shape=jax.ShapeDtypeStruct((B, N, 512), jnp.float32),
    )(nb, c2, c1, W3, W3ms, W3cat, W5)


def kernel(x, W1, A1, W2, A2, W3, W3ms, W3cat, W5):
    B, _, N = x.shape
    coorT = x[:, :12, :]
    coorTp = jnp.concatenate(
        [coorT, jnp.zeros((B, 4, N), coorT.dtype)], axis=1)
    coor_tab = jnp.transpose(coorTp, (0, 2, 1))

    idx1 = _knn(coor_tab, coorTp)
    nb1 = _sc_gather(coor_tab.reshape(B * N, 16), idx1.reshape(-1), ch=2048)
    c1 = _att(nb1.reshape(B, N * _K, 16), coor_tab, W1, A1, C=12)

    idx2 = _knn(c1, jnp.transpose(c1, (0, 2, 1)))
    nb2 = _sc_gather(c1.reshape(B * N, 64), idx2.reshape(-1), ch=1024)
    c2 = _att(nb2.reshape(B, N * _K, 64), c1, W2, A2, C=64)

    idx3 = _knn(c2, jnp.transpose(c2, (0, 2, 1)))
    nb3 = _sc_gather(c2.reshape(B * N, 64), idx3.reshape(-1), ch=1024)
    out = _s3(nb3.reshape(B, N * _K, 64), c2, c1, W3, W3ms, W3cat, W5)

    return jnp.transpose(out, (0, 2, 1))


# speedup vs baseline: 14.9788x; 1.1241x over previous
import functools

import jax
import jax.numpy as jnp
import numpy as np
from jax import lax
from jax.experimental import pallas as pl
from jax.experimental.pallas import tpu as pltpu
from jax.experimental.pallas import tpu_sc as plsc

_K = 16
_BN_S = 1.0 / np.sqrt(1.0 + 1e-5)
_NEG = -3.0e38


def _leaky(y):
    return jnp.where(y > 0, y, 0.2 * y)


def _knn_body(tab_ref, tabT_ref, idx_ref):
    b = pl.program_id(0)
    i = pl.program_id(1)
    xt = tab_ref[0]
    xT = tabT_ref[0]
    n = xT.shape[1]
    tn = xt.shape[0]
    scores = 2.0 * jnp.dot(xt, xT, preferred_element_type=jnp.float32)
    scores = scores - jnp.sum(xT * xT, axis=0, keepdims=True)
    lane = lax.broadcasted_iota(jnp.int32, (tn, n), 1)
    row = lax.broadcasted_iota(jnp.int32, (tn, n), 0) + i * tn
    scores = jnp.where(lane == row, _NEG, scores)
    col = lax.broadcasted_iota(jnp.int32, (tn, _K), 1)
    acc = jnp.zeros((tn, _K), jnp.int32)
    base = b * n
    for j in range(_K):
        sel = jnp.argmax(scores, axis=1).astype(jnp.int32)
        acc = jnp.where(col == j, (sel + base)[:, None], acc)
        scores = jnp.where(lane == sel[:, None], _NEG, scores)
    idx_ref[0] = acc


def _knn(tab, tabT, tn=256):
    B, N, C = tab.shape
    return pl.pallas_call(
        _knn_body,
        grid=(B, N // tn),
        in_specs=[
            pl.BlockSpec((1, tn, C), lambda b, i: (b, i, 0)),
            pl.BlockSpec((1, C, N), lambda b, i: (b, 0, 0)),
        ],
        out_specs=pl.BlockSpec((1, tn, _K), lambda b, i: (b, i, 0)),
        out_shape=jax.ShapeDtypeStruct((B, N, _K), jnp.int32),
    )(tab, tabT)


def _sc_gather(table, idx, ch):
    R, C = table.shape
    Rk = idx.shape[0]
    info = plsc.get_sparse_core_info()
    nc = info.num_cores
    nw = nc * info.num_subcores
    per_w = Rk // nw
    chunks = per_w // ch
    mesh = plsc.VectorSubcoreMesh(core_axis_name="c", subcore_axis_name="s")

    @functools.partial(
        pl.kernel, mesh=mesh,
        out_type=jax.ShapeDtypeStruct((Rk, C), jnp.float32),
        compiler_params=pltpu.CompilerParams(use_tc_tiling_on_sc=False),
        scratch_types=[
            pltpu.VMEM((ch,), jnp.int32),
            pltpu.VMEM((ch, C), jnp.float32),
            pltpu.SemaphoreType.DMA,
        ],
    )
    def k(table_hbm, idx_hbm, out_hbm, idx_v, rows_v, sem):
        wid = lax.axis_index("s") * nc + lax.axis_index("c")
        base = wid * per_w
        for c in range(chunks):
            off = base + c * ch
            pltpu.sync_copy(idx_hbm.at[pl.ds(off, ch)], idx_v)
            pltpu.async_copy(table_hbm.at[idx_v], rows_v, sem).wait()
            pltpu.sync_copy(rows_v, out_hbm.at[pl.ds(off, ch)])

    return k(table, idx)


def _att_body(nb_ref, xc_ref, W_ref, A_ref, out_ref, *, C, Cp):
    nb = nb_ref[0]
    xc = xc_ref[0]
    W = W_ref[...]
    A = A_ref[...]
    Wa, Wb = W[:, :C], W[:, C:]
    Aa, Ab = A[:, :C], A[:, C:]
    if Cp != C:
        z = jnp.zeros((64, Cp - C), jnp.float32)
        Wa = jnp.concatenate([Wa, z], axis=1)
        Wb = jnp.concatenate([Wb, z], axis=1)
        Aa = jnp.concatenate([Aa, z], axis=1)
        Ab = jnp.concatenate([Ab, z], axis=1)
    Wf = jnp.concatenate([Wa, Wb], axis=1)
    Af = jnp.concatenate([Aa, Ab], axis=1)
    tm = xc.shape[0]
    nb3 = nb.reshape(tm, _K, Cp)
    xcb = jnp.broadcast_to(xc[:, None, :], (tm, _K, Cp))
    d = nb3 - xcb
    featW = jnp.concatenate([d, xcb], axis=2).reshape(tm * _K, 2 * Cp)
    featA = jnp.concatenate([xcb - nb3, nb3], axis=2).reshape(tm * _K, 2 * Cp)
    nt = (((1,), (1,)), ((), ()))
    y = lax.dot_general(featW, Wf, nt, preferred_element_type=jnp.float32)
    ey = lax.dot_general(featA, Af, nt, preferred_element_type=jnp.float32)
    cpre = _leaky(y * _BN_S).reshape(tm, _K, 64)
    e = _leaky(ey * _BN_S).reshape(tm, _K, 64)
    m = jnp.max(e, axis=1, keepdims=True)
    w = jnp.exp(e - m)
    att = w / jnp.sum(w, axis=1, keepdims=True)
    out_ref[0] = jnp.sum(att * cpre, axis=1)


def _att(nb, tab, W, A, C, tm=256):
    B, N, Cp = tab.shape
    body = functools.partial(_att_body, C=C, Cp=Cp)
    return pl.pallas_call(
        body,
        grid=(B, N // tm),
        in_specs=[
            pl.BlockSpec((1, tm * _K, Cp), lambda b, i: (b, i, 0)),
            pl.BlockSpec((1, tm, Cp), lambda b, i: (b, i, 0)),
            pl.BlockSpec(W.shape, lambda b, i: (0, 0)),
            pl.BlockSpec(A.shape, lambda b, i: (0, 0)),
        ],
        out_specs=pl.BlockSpec((1, tm, 64), lambda b, i: (b, i, 0)),
        out_shape=jax.ShapeDtypeStruct((B, N, 64), jnp.float32),
    )(nb, tab, W, A)


def _s3_body(nb_ref, c2_ref, c1_ref, W3_ref, W3ms_ref, W3cat_ref, W5_ref,
             out_ref):
    nb = nb_ref[0]
    c2 = c2_ref[0]
    c1 = c1_ref[0]
    W3 = W3_ref[...]
    W3ms = W3ms_ref[...]
    Wst = jnp.concatenate([W3, W3ms], axis=0)
    tm = c2.shape[0]
    nb3 = nb.reshape(tm, _K, 64)
    c2b = jnp.broadcast_to(c2[:, None, :], (tm, _K, 64))
    feat = jnp.concatenate([nb3 - c2b, c2b], axis=2).reshape(tm * _K, 128)
    nt = (((1,), (1,)), ((), ()))
    G = lax.dot_general(feat, Wst, nt, preferred_element_type=jnp.float32)
    c3c = jnp.max(_leaky(G * _BN_S).reshape(tm, _K, 256), axis=1)
    c3cat = _leaky(lax.dot_general(
        c3c, W3cat_ref[...], nt, preferred_element_type=jnp.float32) * _BN_S)
    xall = jnp.concatenate([c1, c2, c3c[:, :128], c3cat], axis=1)
    out_ref[0] = _leaky(lax.dot_general(
        xall, W5_ref[...], nt, preferred_element_type=jnp.float32) * _BN_S)


def _s3(nb, c2, c1, W3, W3ms, W3cat, W5, tm=256):
    B, N, _ = c2.shape
    return pl.pallas_call(
        _s3_body,
        grid=(B, N // tm),
        in_specs=[
            pl.BlockSpec((1, tm * _K, 64), lambda b, i: (b, i, 0)),
            pl.BlockSpec((1, tm, 64), lambda b, i: (b, i, 0)),
            pl.BlockSpec((1, tm, 64), lambda b, i: (b, i, 0)),
            pl.BlockSpec(W3.shape, lambda b, i: (0, 0)),
            pl.BlockSpec(W3ms.shape, lambda b, i: (0, 0)),
            pl.BlockSpec(W3cat.shape, lambda b, i: (0, 0)),
            pl.BlockSpec(W5.shape, lambda b, i: (0, 0)),
        ],
        out_specs=pl.BlockSpec((1, tm, 512), lambda b, i: (b, i, 0)),
        out_shape=jax.ShapeDtypeStruct((B, N, 512), jnp.float32),
    )(nb, c2, c1, W3, W3ms, W3cat, W5)


def kernel(x, W1, A1, W2, A2, W3, W3ms, W3cat, W5):
    B, _, N = x.shape
    coorT = x[:, :12, :]
    coorTp = jnp.concatenate(
        [coorT, jnp.zeros((B, 4, N), coorT.dtype)], axis=1)
    coor_tab = jnp.transpose(coorTp, (0, 2, 1))

    idx1 = _knn(coor_tab, coorTp)
    nb1 = _sc_gather(coor_tab.reshape(B * N, 16), idx1.reshape(-1), ch=2048)
    c1 = _att(nb1.reshape(B, N * _K, 16), coor_tab, W1, A1, C=12)

    idx2 = _knn(c1, jnp.transpose(c1, (0, 2, 1)))
    nb2 = _sc_gather(c1.reshape(B * N, 64), idx2.reshape(-1), ch=1024)
    c2 = _att(nb2.reshape(B, N * _K, 64), c1, W2, A2, C=64)

    idx3 = _knn(c2, jnp.transpose(c2, (0, 2, 1)))
    nb3 = _sc_gather(c2.reshape(B * N, 64), idx3.reshape(-1), ch=1024)
    out = _s3(nb3.reshape(B, N * _K, 64), c2, c1, W3, W3ms, W3cat, W5)

    return jnp.transpose(out, (0, 2, 1))


# speedup vs baseline: 15.3509x; 1.0248x over previous
import functools

import jax
import jax.numpy as jnp
import numpy as np
from jax import lax
from jax.experimental import pallas as pl
from jax.experimental.pallas import tpu as pltpu
from jax.experimental.pallas import tpu_sc as plsc

_K = 16
_BN_S = 1.0 / np.sqrt(1.0 + 1e-5)
_NEG = -3.0e38


def _leaky(y):
    return jnp.where(y > 0, y, 0.2 * y)


def _knn_body(tab_ref, tabT_ref, idx_ref):
    b = pl.program_id(0)
    i = pl.program_id(1)
    xt = tab_ref[0]
    xT = tabT_ref[0]
    n = xT.shape[1]
    tn = xt.shape[0]
    scores = 2.0 * jnp.dot(xt, xT, preferred_element_type=jnp.float32)
    scores = scores - jnp.sum(xT * xT, axis=0, keepdims=True)
    lane = lax.broadcasted_iota(jnp.int32, (tn, n), 1)
    row = lax.broadcasted_iota(jnp.int32, (tn, n), 0) + i * tn
    scores = jnp.where(lane == row, _NEG, scores)
    col = lax.broadcasted_iota(jnp.int32, (tn, _K), 1)
    acc = jnp.zeros((tn, _K), jnp.int32)
    base = b * n
    for j in range(_K):
        sel = jnp.argmax(scores, axis=1).astype(jnp.int32)
        acc = jnp.where(col == j, (sel + base)[:, None], acc)
        scores = jnp.where(lane == sel[:, None], _NEG, scores)
    idx_ref[0] = acc


def _knn(tab, tabT, tn=512):
    B, N, C = tab.shape
    return pl.pallas_call(
        _knn_body,
        grid=(B, N // tn),
        in_specs=[
            pl.BlockSpec((1, tn, C), lambda b, i: (b, i, 0)),
            pl.BlockSpec((1, C, N), lambda b, i: (b, 0, 0)),
        ],
        out_specs=pl.BlockSpec((1, tn, _K), lambda b, i: (b, i, 0)),
        out_shape=jax.ShapeDtypeStruct((B, N, _K), jnp.int32),
    )(tab, tabT)


def _sc_gather(table, idx, ch):
    R, C = table.shape
    Rk = idx.shape[0]
    info = plsc.get_sparse_core_info()
    nc = info.num_cores
    nw = nc * info.num_subcores
    per_w = Rk // nw
    chunks = per_w // ch
    mesh = plsc.VectorSubcoreMesh(core_axis_name="c", subcore_axis_name="s")

    @functools.partial(
        pl.kernel, mesh=mesh,
        out_type=jax.ShapeDtypeStruct((Rk, C), jnp.float32),
        compiler_params=pltpu.CompilerParams(use_tc_tiling_on_sc=False),
        scratch_types=[
            pltpu.VMEM((ch,), jnp.int32),
            pltpu.VMEM((ch,), jnp.int32),
            pltpu.VMEM((ch, C), jnp.float32),
            pltpu.VMEM((ch, C), jnp.float32),
            pltpu.SemaphoreType.DMA,
            pltpu.SemaphoreType.DMA,
        ],
    )
    def k(table_hbm, idx_hbm, out_hbm, idx0, idx1, rows0, rows1, sem0, sem1):
        wid = lax.axis_index("s") * nc + lax.axis_index("c")
        base = wid * per_w
        idx_v = (idx0, idx1)
        rows_v = (rows0, rows1)
        sems = (sem0, sem1)
        pltpu.sync_copy(idx_hbm.at[pl.ds(base, ch)], idx0)
        cp0 = pltpu.async_copy(table_hbm.at[idx0], rows0, sem0)
        for c in range(chunks):
            p, q = c % 2, (c + 1) % 2
            if c + 1 < chunks:
                off_n = base + (c + 1) * ch
                pltpu.sync_copy(idx_hbm.at[pl.ds(off_n, ch)], idx_v[q])
                nxt = pltpu.async_copy(table_hbm.at[idx_v[q]], rows_v[q],
                                       sems[q])
            cp0.wait()
            pltpu.sync_copy(rows_v[p], out_hbm.at[pl.ds(base + c * ch, ch)])
            if c + 1 < chunks:
                cp0 = nxt

    return k(table, idx)


def _att_body(nb_ref, xc_ref, W_ref, A_ref, out_ref, *, C, Cp):
    nb = nb_ref[0]
    xc = xc_ref[0]
    W = W_ref[...]
    A = A_ref[...]
    Wa, Wb = W[:, :C], W[:, C:]
    Aa, Ab = A[:, :C], A[:, C:]
    if Cp != C:
        z = jnp.zeros((64, Cp - C), jnp.float32)
        Wa = jnp.concatenate([Wa, z], axis=1)
        Wb = jnp.concatenate([Wb, z], axis=1)
        Aa = jnp.concatenate([Aa, z], axis=1)
        Ab = jnp.concatenate([Ab, z], axis=1)
    Wf = jnp.concatenate([Wa, Wb], axis=1)
    Af = jnp.concatenate([Aa, Ab], axis=1)
    tm = xc.shape[0]
    nb3 = nb.reshape(tm, _K, Cp)
    xcb = jnp.broadcast_to(xc[:, None, :], (tm, _K, Cp))
    d = nb3 - xcb
    featW = jnp.concatenate([d, xcb], axis=2).reshape(tm * _K, 2 * Cp)
    featA = jnp.concatenate([xcb - nb3, nb3], axis=2).reshape(tm * _K, 2 * Cp)
    nt = (((1,), (1,)), ((), ()))
    y = lax.dot_general(featW, Wf, nt, preferred_element_type=jnp.float32)
    ey = lax.dot_general(featA, Af, nt, preferred_element_type=jnp.float32)
    cpre = _leaky(y * _BN_S).reshape(tm, _K, 64)
    e = _leaky(ey * _BN_S).reshape(tm, _K, 64)
    m = jnp.max(e, axis=1, keepdims=True)
    w = jnp.exp(e - m)
    att = w / jnp.sum(w, axis=1, keepdims=True)
    out_ref[0] = jnp.sum(att * cpre, axis=1)


def _att(nb, tab, W, A, C, tm=256):
    B, N, Cp = tab.shape
    body = functools.partial(_att_body, C=C, Cp=Cp)
    return pl.pallas_call(
        body,
        grid=(B, N // tm),
        in_specs=[
            pl.BlockSpec((1, tm * _K, Cp), lambda b, i: (b, i, 0)),
            pl.BlockSpec((1, tm, Cp), lambda b, i: (b, i, 0)),
            pl.BlockSpec(W.shape, lambda b, i: (0, 0)),
            pl.BlockSpec(A.shape, lambda b, i: (0, 0)),
        ],
        out_specs=pl.BlockSpec((1, tm, 64), lambda b, i: (b, i, 0)),
        out_shape=jax.ShapeDtypeStruct((B, N, 64), jnp.float32),
    )(nb, tab, W, A)


def _s3_body(nb_ref, c2_ref, c1_ref, W3_ref, W3ms_ref, W3cat_ref, W5_ref,
             out_ref):
    nb = nb_ref[0]
    c2 = c2_ref[0]
    c1 = c1_ref[0]
    W3 = W3_ref[...]
    W3ms = W3ms_ref[...]
    Wst = jnp.concatenate([W3, W3ms], axis=0)
    tm = c2.shape[0]
    nb3 = nb.reshape(tm, _K, 64)
    c2b = jnp.broadcast_to(c2[:, None, :], (tm, _K, 64))
    feat = jnp.concatenate([nb3 - c2b, c2b], axis=2).reshape(tm * _K, 128)
    nt = (((1,), (1,)), ((), ()))
    G = lax.dot_general(feat, Wst, nt, preferred_element_type=jnp.float32)
    c3c = jnp.max(_leaky(G * _BN_S).reshape(tm, _K, 256), axis=1)
    c3cat = _leaky(lax.dot_general(
        c3c, W3cat_ref[...], nt, preferred_element_type=jnp.float32) * _BN_S)
    xall = jnp.concatenate([c1, c2, c3c[:, :128], c3cat], axis=1)
    out_ref[0] = _leaky(lax.dot_general(
        xall, W5_ref[...], nt, preferred_element_type=jnp.float32) * _BN_S)


def _s3(nb, c2, c1, W3, W3ms, W3cat, W5, tm=256):
    B, N, _ = c2.shape
    return pl.pallas_call(
        _s3_body,
        grid=(B, N // tm),
        in_specs=[
            pl.BlockSpec((1, tm * _K, 64), lambda b, i: (b, i, 0)),
            pl.BlockSpec((1, tm, 64), lambda b, i: (b, i, 0)),
            pl.BlockSpec((1, tm, 64), lambda b, i: (b, i, 0)),
            pl.BlockSpec(W3.shape, lambda b, i: (0, 0)),
            pl.BlockSpec(W3ms.shape, lambda b, i: (0, 0)),
            pl.BlockSpec(W3cat.shape, lambda b, i: (0, 0)),
            pl.BlockSpec(W5.shape, lambda b, i: (0, 0)),
        ],
        out_specs=pl.BlockSpec((1, tm, 512), lambda b, i: (b, i, 0)),
        out_shape=jax.ShapeDtypeStruct((B, N, 512), jnp.float32),
    )(nb, c2, c1, W3, W3ms, W3cat, W5)


def kernel(x, W1, A1, W2, A2, W3, W3ms, W3cat, W5):
    B, _, N = x.shape
    coorT = x[:, :12, :]
    coorTp = jnp.concatenate(
        [coorT, jnp.zeros((B, 4, N), coorT.dtype)], axis=1)
    coor_tab = jnp.transpose(coorTp, (0, 2, 1))

    idx1 = _knn(coor_tab, coorTp)
    nb1 = _sc_gather(coor_tab.reshape(B * N, 16), idx1.reshape(-1), ch=1024)
    c1 = _att(nb1.reshape(B, N * _K, 16), coor_tab, W1, A1, C=12)

    idx2 = _knn(c1, jnp.transpose(c1, (0, 2, 1)))
    nb2 = _sc_gather(c1.reshape(B * N, 64), idx2.reshape(-1), ch=512)
    c2 = _att(nb2.reshape(B, N * _K, 64), c1, W2, A2, C=64)

    idx3 = _knn(c2, jnp.transpose(c2, (0, 2, 1)))
    nb3 = _sc_gather(c2.reshape(B * N, 64), idx3.reshape(-1), ch=512)
    out = _s3(nb3.reshape(B, N * _K, 64), c2, c1, W3, W3ms, W3cat, W5)

    return jnp.transpose(out, (0, 2, 1))
